# Initial kernel scaffold; baseline (speedup 1.0000x reference)
import jax
import jax.numpy as jnp
from jax.experimental import pallas as pl


def kernel(x, conv0a_w, conv0a_b, conv0b_w, conv0b_b, bn0_g, bn0_b, conv1a_w, conv1a_b, conv1b_w, conv1b_b, bn1_g, bn1_b, conv2a_w, conv2a_b, conv2b_w, conv2b_b, bn2_g, bn2_b, conv3a_w, conv3a_b, conv3b_w, conv3b_b, bn3_g, bn3_b, fc1_w, fc1_b, fc2_w, fc2_b, fc3_w, fc3_b, fc4_w, fc4_b):
    raise NotImplementedError("write your pallas kernel here")


# speedup vs baseline: 1.3011x; 1.3011x over previous
import functools

import jax
import jax.numpy as jnp
from jax import lax
from jax.experimental import pallas as pl
from jax.experimental.pallas import tpu as pltpu

_CPAD = 16
_CONV_CH = (16, 32, 32, 64)


def _rup(a, b):
    return -(-a // b) * b


class _Lay:
    def __init__(self, N, D, H, W, cpi_max):
        self.N, self.D, self.H, self.W = N, D, H, W
        self.Dp, self.Hp, self.Wp = D + 2, H + 2, W + 2
        self.Mp = N * self.Dp * self.Hp * self.Wp
        G = self.Hp * self.Wp + self.Wp + 1
        self.hal = _rup(G, 128)
        patch_cap = (12 * 1024 * 1024) // (27 * cpi_max * 2)
        k = max(1, min(patch_cap // self.hal, 13440 // self.hal,
                       max(1, self.Mp // (2 * self.hal))))
        self.tm = k * self.hal
        self.T = -(-self.Mp // self.tm)
        self.Mp_c = self.T * self.tm
        self.front = self.tm
        self.L = self.Mp_c + 2 * self.tm
        self.offs27 = tuple(dz * self.Hp * self.Wp + dy * self.Wp + dx
                            for dz in (-1, 0, 1) for dy in (-1, 0, 1)
                            for dx in (-1, 0, 1))
        self.offs9 = tuple(dz * self.Hp * self.Wp + dy * self.Wp
                           for dz in (-1, 0, 1) for dy in (-1, 0, 1))


def _interior_mask(lay):
    m = jnp.pad(jnp.ones((lay.N, lay.D, lay.H, lay.W), jnp.float32),
                ((0, 0), (1, 1), (1, 1), (1, 1)))
    m = m.reshape(1, lay.Mp)
    m = jnp.pad(m, ((0, 0), (0, lay.Mp_c - lay.Mp)))
    return m.astype(jnp.bfloat16)


def _embed_flat(v, lay):
    C = v.shape[0]
    vp = jnp.pad(v, ((0, 0), (0, 0), (1, 1), (1, 1), (1, 1)))
    flat = vp.reshape(C, lay.Mp)
    flat = jnp.pad(flat, ((0, 0), (lay.front, lay.L - lay.front - lay.Mp)))
    return flat.astype(jnp.bfloat16)


def _embed_shifted16(x, lay):
    v = jnp.transpose(x, (1, 0, 2, 3, 4)).astype(jnp.bfloat16)
    vp = jnp.pad(v, ((0, 0), (0, 0), (1, 1), (1, 1), (1, 1)))
    flat = vp.reshape(1, lay.Mp)
    flat = jnp.pad(flat, ((0, 0),
                          (lay.front + 8, lay.L - lay.front - lay.Mp + 8)))
    rows = [lax.dynamic_slice(flat, (0, r), (1, lay.L)) for r in range(16)]
    return jnp.concatenate(rows, axis=0)


def _conv_body(xl_ref, xc_ref, xr_ref, wt_ref, b_ref, m_ref, out_ref, *rest,
               offsets, hal, zero_edges, stats):
    if stats:
        stats_ref, scr = rest
    else:
        (scr,) = rest
    cpi, tm = xc_ref.shape
    cpo = wt_ref.shape[1]

    scr[:, :hal] = xl_ref[...]
    scr[:, hal:hal + tm] = xc_ref[...]
    scr[:, hal + tm:] = xr_ref[...]

    if zero_edges:
        @pl.when(pl.program_id(0) == 0)
        def _():
            scr[:, :hal] = jnp.zeros((cpi, hal), scr.dtype)

        @pl.when(pl.program_id(0) == pl.num_programs(0) - 1)
        def _():
            scr[:, hal + tm:] = jnp.zeros((cpi, hal), scr.dtype)

    taps = [scr[:, hal + o:hal + o + tm] for o in offsets]
    patches = jnp.concatenate(taps, axis=0)

    yt = lax.dot_general(patches, wt_ref[...], (((0,), (0,)), ((), ())),
                         preferred_element_type=jnp.float32)
    yt = jnp.maximum(yt + b_ref[...], 0.0)
    yb = yt.astype(jnp.bfloat16).T
    yb = yb * m_ref[...]
    out_ref[...] = yb

    if stats:
        yf = yb.astype(jnp.float32)
        s = jnp.sum(yf, axis=1, keepdims=True)
        sq = jnp.sum(yf * yf, axis=1, keepdims=True)
        lane = lax.broadcasted_iota(jnp.int32, (cpo, 128), 1)
        stats_ref[0] = (jnp.where(lane == 0, s, 0.0)
                        + jnp.where(lane == 1, sq, 0.0))


def _conv(x_flat, wt, b_row, mask, lay, *, offsets, zero_edges, stats):
    cpi, L = x_flat.shape
    K, cpo = wt.shape
    tm, hal, T = lay.tm, lay.hal, lay.T
    r = tm // hal

    body = functools.partial(_conv_body, offsets=offsets, hal=hal,
                             zero_edges=zero_edges, stats=stats)
    out_shape = jax.ShapeDtypeStruct((cpo, L), jnp.bfloat16)
    out_specs = pl.BlockSpec((cpo, tm), lambda i: (0, 1 + i))
    if stats:
        out_shape = (out_shape,
                     jax.ShapeDtypeStruct((T, cpo, 128), jnp.float32))
        out_specs = (out_specs, pl.BlockSpec((1, cpo, 128), lambda i: (i, 0, 0)))

    return pl.pallas_call(
        body,
        out_shape=out_shape,
        grid=(T,),
        in_specs=[
            pl.BlockSpec((cpi, hal), lambda i: (0, r * (i + 1) - 1)),
            pl.BlockSpec((cpi, tm), lambda i: (0, 1 + i)),
            pl.BlockSpec((cpi, hal), lambda i: (0, r * (i + 2))),
            pl.BlockSpec((K, cpo), lambda i: (0, 0)),
            pl.BlockSpec((1, cpo), lambda i: (0, 0)),
            pl.BlockSpec((1, tm), lambda i: (0, i)),
        ],
        out_specs=out_specs,
        scratch_shapes=[pltpu.VMEM((cpi, tm + 2 * hal), jnp.bfloat16)],
        compiler_params=pltpu.CompilerParams(
            dimension_semantics=("parallel",),
            vmem_limit_bytes=56 * 1024 * 1024,
        ),
    )(x_flat, x_flat, x_flat, wt, b_row, mask)


def _mlp_body(x_ref, w1_ref, b1_ref, w2_ref, b2_ref, w3_ref, b3_ref,
              w4_ref, b4_ref, o_ref):
    dn = (((0,), (0,)), ((), ()))
    h = x_ref[...]
    h = jnp.maximum(lax.dot_general(w1_ref[...], h, dn,
                                    preferred_element_type=jnp.float32)
                    + b1_ref[...], 0.0)
    h = jnp.maximum(lax.dot_general(w2_ref[...], h, dn,
                                    preferred_element_type=jnp.float32)
                    + b2_ref[...], 0.0)
    h = jnp.maximum(lax.dot_general(w3_ref[...], h, dn,
                                    preferred_element_type=jnp.float32)
                    + b3_ref[...], 0.0)
    z = lax.dot_general(w4_ref[...], h, dn,
                        preferred_element_type=jnp.float32) + b4_ref[...]
    o_ref[...] = jax.nn.sigmoid(z)


def _mlp_head(feats_t, params):
    args = (feats_t,
            params["fc1_w"], params["fc1_b"].T,
            params["fc2_w"], params["fc2_b"].T,
            params["fc3_w"], params["fc3_b"].T,
            params["fc4_w"], params["fc4_b"].T)
    n = feats_t.shape[1]
    n_cls = params["fc4_w"].shape[1]
    out = pl.pallas_call(
        _mlp_body,
        out_shape=jax.ShapeDtypeStruct((n_cls, n), jnp.float32),
        grid=(1,),
        in_specs=[pl.BlockSpec(a.shape, lambda i, n=len(a.shape): (0,) * n)
                  for a in args],
        out_specs=pl.BlockSpec((n_cls, n), lambda i: (0, 0)),
        compiler_params=pltpu.CompilerParams(
            dimension_semantics=("arbitrary",)),
    )(*args)
    return out.T


def _forward(x, params):
    N, _, D, H, W = x.shape

    v = None
    for blk in range(4):
        wa, ba = params[f"conv{blk}a_w"], params[f"conv{blk}a_b"]
        wb, bb = params[f"conv{blk}b_w"], params[f"conv{blk}b_b"]
        cpi_a = wa.shape[1] // 27
        cpi_b = wb.shape[1] // 27
        cpo = wb.shape[0]
        lay = _Lay(N, D, H, W, cpi_b)
        mask = _interior_mask(lay)

        if blk == 0:
            x_flat = _embed_shifted16(x, lay)
            wa_taps = wa.reshape(cpo, 27, cpi_a)[:, :, 0].reshape(cpo, 9, 3)
            w9 = jnp.transpose(wa_taps, (1, 2, 0))
            w9 = jnp.pad(w9, ((0, 0), (7, 6), (0, 0)))
            wt_a = w9.reshape(144, cpo).astype(jnp.bfloat16)
            offs_a = lay.offs9
            zero_a = False
        else:
            x_flat = _embed_flat(v, lay)
            wt_a = wa.T
            offs_a = lay.offs27
            zero_a = False

        y = _conv(x_flat, wt_a, ba.T, mask, lay, offsets=offs_a,
                  zero_edges=zero_a, stats=False)
        y, st = _conv(y, wb.T, bb.T, mask, lay, offsets=lay.offs27,
                      zero_edges=True, stats=True)

        cnt = jnp.float32(N * D * H * W)
        s = jnp.sum(st[:, :, 0], axis=0)
        sq = jnp.sum(st[:, :, 1], axis=0)
        mean = s / cnt
        var = jnp.maximum(sq / cnt - mean * mean, 0.0)
        scale = params[f"bn{blk}_g"] * lax.rsqrt(var + 1e-5)
        shift = params[f"bn{blk}_b"] - mean * scale

        core = y[:, lay.front:lay.front + lay.Mp]
        core = core.reshape(cpo, N, lay.Dp, lay.Hp, lay.Wp)[
            :, :, 1:-1, 1:-1, 1:-1]
        r8 = core.reshape(cpo, N, D // 2, 2, H // 2, 2, W // 2, 2)
        pmax = jnp.max(r8, axis=(3, 5, 7)).astype(jnp.float32)
        pmin = jnp.min(r8, axis=(3, 5, 7)).astype(jnp.float32)
        sc = scale[:, None, None, None, None]
        sh = shift[:, None, None, None, None]
        v = jnp.where(sc >= 0, pmax, pmin) * sc + sh
        D, H, W = D // 2, H // 2, W // 2

    nch = _CONV_CH[3]
    feats_t = jnp.transpose(v[:nch], (0, 2, 3, 4, 1)).reshape(-1, N)
    feats_t = feats_t.astype(jnp.float32)
    return _mlp_head(feats_t, params)


def kernel(x,
           conv0a_w, conv0a_b, conv0b_w, conv0b_b, bn0_g, bn0_b,
           conv1a_w, conv1a_b, conv1b_w, conv1b_b, bn1_g, bn1_b,
           conv2a_w, conv2a_b, conv2b_w, conv2b_b, bn2_g, bn2_b,
           conv3a_w, conv3a_b, conv3b_w, conv3b_b, bn3_g, bn3_b,
           fc1_w, fc1_b, fc2_w, fc2_b, fc3_w, fc3_b, fc4_w, fc4_b):
    params = {
        "conv0a_w": conv0a_w, "conv0a_b": conv0a_b,
        "conv0b_w": conv0b_w, "conv0b_b": conv0b_b,
        "bn0_g": bn0_g, "bn0_b": bn0_b,
        "conv1a_w": conv1a_w, "conv1a_b": conv1a_b,
        "conv1b_w": conv1b_w, "conv1b_b": conv1b_b,
        "bn1_g": bn1_g, "bn1_b": bn1_b,
        "conv2a_w": conv2a_w, "conv2a_b": conv2a_b,
        "conv2b_w": conv2b_w, "conv2b_b": conv2b_b,
        "bn2_g": bn2_g, "bn2_b": bn2_b,
        "conv3a_w": conv3a_w, "conv3a_b": conv3a_b,
        "conv3b_w": conv3b_w, "conv3b_b": conv3b_b,
        "bn3_g": bn3_g, "bn3_b": bn3_b,
        "fc1_w": fc1_w, "fc1_b": fc1_b, "fc2_w": fc2_w, "fc2_b": fc2_b,
        "fc3_w": fc3_w, "fc3_b": fc3_b, "fc4_w": fc4_w, "fc4_b": fc4_b,
    }
    return _forward(x, params)
